# baseline (speedup 1.0000x reference)
import functools

import jax
import jax.numpy as jnp
import numpy as np
from jax import lax
from jax.experimental import pallas as pl
from jax.experimental.pallas import tpu as pltpu
from jax.experimental.pallas import tpu_sc as plsc

B = 4096
L = 200
D = 32
V = 1000000
NW = 32
CHUNK = 1024
NQ = B // CHUNK
ITEMS = L * NQ
PER_W = ITEMS // NW
VFULL = V // 128
VTAIL = V - VFULL * 128
PREP_K = (VFULL + NW - 1) // NW

_mesh = plsc.VectorSubcoreMesh(core_axis_name="c", subcore_axis_name="s")
_IOTA = np.arange(16, dtype=np.int32)


@functools.partial(
    pl.kernel,
    out_type=(
        jax.ShapeDtypeStruct((V // 4, 128), jnp.float32),
        jax.ShapeDtypeStruct((L, B // 128, 128), jnp.int32),
    ),
    mesh=_mesh,
    scratch_types=[
        pltpu.VMEM((32, 128), jnp.float32),
        pltpu.VMEM((32, 128), jnp.float32),
        pltpu.VMEM((8, 128), jnp.int32),
    ],
    compiler_params=pltpu.CompilerParams(use_tc_tiling_on_sc=True,
                                         needs_layout_passes=False),
)
def _prep(xt_hbm, tabt_hbm, tail_hbm, trm_hbm, xi_hbm, tin_v, tout_v, xb_v):
    wid = lax.axis_index("s") * 2 + lax.axis_index("c")
    iota = lax.iota(jnp.int32, 16)

    def transpose_block():
        def body(t, _):
            col = jnp.full((16,), t, jnp.int32)
            lo = plsc.load_gather(tin_v, [iota, col])
            hi = plsc.load_gather(tin_v, [iota + 16, col])
            r = t // 4
            c0 = (t % 4) * 32
            tout_v[r, pl.ds(c0, 16)] = lo
            tout_v[r, pl.ds(c0 + 16, 16)] = hi
            return 0
        lax.fori_loop(0, 128, body, 0)

    def prep_body(k, carry):
        b = wid + NW * k

        @pl.when(b < VFULL)
        def _full():
            pltpu.sync_copy(tabt_hbm.at[:, pl.ds(b * 128, 128)], tin_v)
            transpose_block()
            pltpu.sync_copy(tout_v, trm_hbm.at[pl.ds(b * 32, 32)])

        return carry

    lax.fori_loop(0, PREP_K, prep_body, 0)

    @pl.when(wid == 0)
    def _tail():
        pltpu.sync_copy(tail_hbm, tout_v.at[pl.ds(0, VTAIL * D // 128)])
        pltpu.sync_copy(tout_v.at[pl.ds(0, VTAIL * D // 128)],
                        trm_hbm.at[pl.ds(VFULL * 32, VTAIL * D // 128)])

    def x_body(j, carry):
        m = wid * PER_W + j
        lt = m // (B // 128)
        c = m % (B // 128)
        pltpu.sync_copy(xt_hbm.at[pl.ds(lt * 8, 8), pl.ds(c * 128, 128)], xb_v)
        pltpu.sync_copy(xb_v, xi_hbm.at[pl.ds(lt * 8, 8), c])
        return carry

    lax.fori_loop(0, PER_W, x_body, 0)


@functools.partial(
    pl.kernel,
    out_type=jax.ShapeDtypeStruct((L, D // 8, B // 128, 8, 128), jnp.float32),
    mesh=_mesh,
    scratch_types=[
        pltpu.VMEM((8, 128), jnp.int32),
        pltpu.VMEM((CHUNK, D), jnp.float32),
        pltpu.VMEM((4, 128), jnp.float32),
        pltpu.VMEM((D // 8, CHUNK // 128, 8, 128), jnp.float32),
        pltpu.SemaphoreType.DMA,
    ],
    compiler_params=pltpu.CompilerParams(use_tc_tiling_on_sc=False,
                                         needs_layout_passes=False),
)
def _lookup(xi_hbm, trm_hbm, posb_hbm, out_hbm,
            idx_v, rows_v, pos_v, ob_v, gsem):
    wid = lax.axis_index("s") * 2 + lax.axis_index("c")
    iota = lax.iota(jnp.int32, 16)

    def item_body(j, carry):
        m = wid * PER_W + j
        l = m // NQ
        q = m % NQ
        pltpu.sync_copy(xi_hbm.at[l, pl.ds(q * 8, 8)], idx_v)
        pltpu.sync_copy(posb_hbm.at[pl.ds(l * 4, 4)], pos_v)
        descs = [
            pltpu.async_copy(trm_hbm.at[idx_v.at[k]],
                             rows_v.at[pl.ds(k * 128, 128)], gsem)
            for k in range(8)
        ]
        for dsc in descs:
            dsc.wait()

        for d in range(D):
            r, s = d // 8, d % 8
            g, kk = d // 8, d % 8
            splat = pos_v[g, pl.ds(kk * 16, 16)]
            def col_body(jj, _, r=r, s=s, splat=splat):
                vals = plsc.load_gather(rows_v,
                                        [jj * 16 + iota, jnp.full((16,), d, jnp.int32)])
                cp = jj // 8
                mm = jj % 8
                ob_v[r, cp, s, pl.ds(mm * 16, 16)] = vals + splat
                return 0

            lax.fori_loop(0, CHUNK // 16, col_body, 0)

        pltpu.sync_copy(ob_v, out_hbm.at[l, :, pl.ds(q * (CHUNK // 128), CHUNK // 128)])
        return carry

    lax.fori_loop(0, PER_W, item_body, 0)


def kernel(x, embedding_table, possitional_emb):
    xt = x.T.astype(jnp.int32)
    tabt = embedding_table.T
    tail = embedding_table[VFULL * 128:].reshape(VTAIL * D // 128, 128)
    posb = (jnp.broadcast_to(possitional_emb[:, :, None], (L, D, 16))
            .reshape(L * 4, 128))
    trm, xi = _prep(xt, tabt, tail)
    out5 = _lookup(xi, trm.reshape(V, D), posb)
    return out5.transpose(2, 4, 0, 1, 3).reshape(B, L, D)


# speedup vs baseline: 1.2422x; 1.2422x over previous
import functools

import jax
import jax.numpy as jnp
from jax import lax
from jax.experimental import pallas as pl
from jax.experimental.pallas import tpu as pltpu
from jax.experimental.pallas import tpu_sc as plsc

B = 4096
L = 200
D = 32
V = 1000000
NW = 32
CHUNK = 512
NQ = B // CHUNK
ITEMS = L * NQ
PER_W = ITEMS // NW
NG = CHUNK // 128
VFULL = V // 128
VMAIN = (VFULL // NW) * NW
KMAIN = VMAIN // NW
VTAIL = V - VFULL * 128

_mesh = plsc.VectorSubcoreMesh(core_axis_name="c", subcore_axis_name="s")


@functools.partial(
    pl.kernel,
    out_type=(
        jax.ShapeDtypeStruct((V // 4, 128), jnp.float32),
        jax.ShapeDtypeStruct((L, B // 128, 128), jnp.int32),
    ),
    mesh=_mesh,
    scratch_types=[
        pltpu.VMEM((2, 32, 128), jnp.float32),
        pltpu.VMEM((2, 32, 128), jnp.float32),
        pltpu.VMEM((PER_W // 2, 8, 128), jnp.int32),
        pltpu.SemaphoreType.DMA,
        pltpu.SemaphoreType.DMA,
        pltpu.SemaphoreType.DMA,
        pltpu.SemaphoreType.DMA,
        pltpu.SemaphoreType.DMA,
        pltpu.SemaphoreType.DMA,
    ],
    compiler_params=pltpu.CompilerParams(use_tc_tiling_on_sc=True,
                                         needs_layout_passes=False),
)
def _prep(xt_hbm, tabt_hbm, tail_hbm, trm_hbm, xi_hbm,
          tin_v, tout_v, xb_v, isemA, isemB, osemA, osemB, xisem, xosem):
    wid = lax.axis_index("s") * 2 + lax.axis_index("c")
    iota = lax.iota(jnp.int32, 16)

    def transpose_block(src, dst):
        def body(t, _):
            col = jnp.full((16,), t, jnp.int32)
            lo = plsc.load_gather(src, [iota, col])
            hi = plsc.load_gather(src, [iota + 16, col])
            r = t // 4
            c0 = (t % 4) * 32
            dst[r, pl.ds(c0, 16)] = lo
            dst[r, pl.ds(c0 + 16, 16)] = hi
            return 0
        lax.fori_loop(0, 128, body, 0, unroll=8)

    def start_in(k, buf):
        b = wid + NW * k
        pltpu.async_copy(tabt_hbm.at[:, pl.ds(b * 128, 128)],
                         tin_v.at[buf], isemA if buf == 0 else isemB)

    def wait_in(buf):
        pltpu.make_async_copy(tabt_hbm.at[:, pl.ds(0, 128)],
                              tin_v.at[buf],
                              isemA if buf == 0 else isemB).wait()

    def start_out(k, buf):
        b = wid + NW * k
        pltpu.async_copy(tout_v.at[buf], trm_hbm.at[pl.ds(b * 32, 32)],
                         osemA if buf == 0 else osemB)

    def wait_out(k, buf):
        b = wid + NW * k
        pltpu.make_async_copy(tout_v.at[buf], trm_hbm.at[pl.ds(b * 32, 32)],
                              osemA if buf == 0 else osemB).wait()

    start_in(0, 0)

    def pair_body(kk, carry):
        for par in (0, 1):
            k = kk * 2 + par
            wait_in(par)

            @pl.when(k + 1 < KMAIN)
            def _():
                start_in(k + 1, 1 - par)

            @pl.when(k >= 2)
            def _():
                wait_out(k - 2, par)

            transpose_block(tin_v.at[par], tout_v.at[par])
            start_out(k, par)
        return carry

    lax.fori_loop(0, KMAIN // 2, pair_body, 0)
    wait_out(KMAIN - 2, 0)
    wait_out(KMAIN - 1, 1)

    @pl.when(wid < VFULL - VMAIN)
    def _ragged():
        b = VMAIN + wid
        pltpu.sync_copy(tabt_hbm.at[:, pl.ds(b * 128, 128)], tin_v.at[0])
        transpose_block(tin_v.at[0], tout_v.at[0])
        pltpu.sync_copy(tout_v.at[0], trm_hbm.at[pl.ds(b * 32, 32)])

    @pl.when(wid == VFULL - VMAIN)
    def _tail():
        pltpu.sync_copy(tail_hbm, tout_v.at[0, pl.ds(0, VTAIL * D // 128)])
        pltpu.sync_copy(tout_v.at[0, pl.ds(0, VTAIL * D // 128)],
                        trm_hbm.at[pl.ds(VFULL * 32, VTAIL * D // 128)])

    NB = PER_W // 2
    for i in range(NB):
        m = wid * NB + i
        lt = m // (B // 128)
        c = m % (B // 128)
        pltpu.async_copy(xt_hbm.at[pl.ds(lt * 8, 8), pl.ds(c * 128, 128)],
                         xb_v.at[i], xisem)
    for i in range(NB):
        pltpu.make_async_copy(xt_hbm.at[pl.ds(0, 8), pl.ds(0, 128)],
                              xb_v.at[i], xisem).wait()
    for i in range(NB):
        m = wid * NB + i
        lt = m // (B // 128)
        c = m % (B // 128)
        pltpu.async_copy(xb_v.at[i], xi_hbm.at[pl.ds(lt * 8, 8), c], xosem)
    for i in range(NB):
        m = wid * NB + i
        lt = m // (B // 128)
        c = m % (B // 128)
        pltpu.make_async_copy(xb_v.at[i], xi_hbm.at[pl.ds(lt * 8, 8), c],
                              xosem).wait()


@functools.partial(
    pl.kernel,
    out_type=jax.ShapeDtypeStruct((L, D // 8, B // 128, 8, 128), jnp.float32),
    mesh=_mesh,
    scratch_types=[
        pltpu.VMEM((2, NG, 128), jnp.int32),
        pltpu.VMEM((2, CHUNK, D), jnp.float32),
        pltpu.VMEM((2, 4, 128), jnp.float32),
        pltpu.VMEM((2, D // 8, NG, 8, 128), jnp.float32),
        pltpu.SemaphoreType.DMA,
        pltpu.SemaphoreType.DMA,
        pltpu.SemaphoreType.DMA,
        pltpu.SemaphoreType.DMA,
        pltpu.SemaphoreType.DMA,
        pltpu.SemaphoreType.DMA,
    ],
    compiler_params=pltpu.CompilerParams(use_tc_tiling_on_sc=False,
                                         needs_layout_passes=False),
)
def _lookup(xi_hbm, trm_hbm, posb_hbm, out_hbm,
            idx_v, rows_v, pos_v, ob_v,
            isemA, isemB, gsemA, gsemB, osemA, osemB):
    wid = lax.axis_index("s") * 2 + lax.axis_index("c")
    iota = lax.iota(jnp.int32, 16)
    dconst = [jnp.full((16,), d, jnp.int32) for d in range(D)]
    isem = (isemA, isemB)
    gsem = (gsemA, gsemB)
    osem = (osemA, osemB)

    def lq(j):
        m = wid * PER_W + j
        return m // NQ, m % NQ

    def start_inputs(j, par):
        l, q = lq(j)
        pltpu.async_copy(xi_hbm.at[l, pl.ds(q * NG, NG)], idx_v.at[par],
                         isem[par])
        pltpu.async_copy(posb_hbm.at[pl.ds(l * 4, 4)], pos_v.at[par],
                         isem[par])

    def wait_inputs(par):
        pltpu.make_async_copy(xi_hbm.at[0, pl.ds(0, NG)], idx_v.at[par],
                              isem[par]).wait()
        pltpu.make_async_copy(posb_hbm.at[pl.ds(0, 4)], pos_v.at[par],
                              isem[par]).wait()

    def start_gathers(par):
        for k in range(NG):
            pltpu.async_copy(trm_hbm.at[idx_v.at[par, k]],
                             rows_v.at[par, pl.ds(k * 128, 128)], gsem[par])

    def wait_gathers(par):
        pltpu.make_async_copy(trm_hbm.at[pl.ds(0, CHUNK)], rows_v.at[par],
                              gsem[par]).wait()

    def start_write(j, par):
        l, q = lq(j)
        pltpu.async_copy(ob_v.at[par],
                         out_hbm.at[l, :, pl.ds(q * NG, NG)], osem[par])

    def wait_write(j, par):
        l, q = lq(j)
        pltpu.make_async_copy(ob_v.at[par],
                              out_hbm.at[l, :, pl.ds(q * NG, NG)],
                              osem[par]).wait()

    def compute(par):
        rows = rows_v.at[par]
        for dq in range(D // 8):
            splats = [pos_v[par, dq, pl.ds(k * 16, 16)] for k in range(8)]

            def jj_body(jj, _, dq=dq, splats=splats):
                row_idx = jj * 16 + iota
                cp = jj // 8
                mm = jj % 8
                for k in range(8):
                    vals = plsc.load_gather(rows, [row_idx, dconst[dq * 8 + k]])
                    ob_v[par, dq, cp, k, pl.ds(mm * 16, 16)] = vals + splats[k]
                return 0

            lax.fori_loop(0, CHUNK // 16, jj_body, 0, unroll=4)

    start_inputs(0, 0)
    wait_inputs(0)
    start_gathers(0)
    start_inputs(1, 1)

    def pair_body(kk, carry):
        for par in (0, 1):
            j = kk * 2 + par

            @pl.when(j + 1 < PER_W)
            def _():
                wait_inputs(1 - par)
                start_gathers(1 - par)

            wait_gathers(par)

            @pl.when(j >= 2)
            def _():
                wait_write(j - 2, par)

            compute(par)
            start_write(j, par)

            @pl.when(j + 2 < PER_W)
            def _():
                start_inputs(j + 2, par)
        return carry

    lax.fori_loop(0, PER_W // 2, pair_body, 0)
    wait_write(PER_W - 2, 0)
    wait_write(PER_W - 1, 1)


def kernel(x, embedding_table, possitional_emb):
    xt = x.T.astype(jnp.int32)
    tabt = embedding_table.T
    tail = embedding_table[VFULL * 128:].reshape(VTAIL * D // 128, 128)
    posb = (jnp.broadcast_to(possitional_emb[:, :, None], (L, D, 16))
            .reshape(L * 4, 128))
    trm, xi = _prep(xt, tabt, tail)
    out5 = _lookup(xi, trm.reshape(V, D), posb)
    return out5.transpose(2, 4, 0, 1, 3).reshape(B, L, D)


# speedup vs baseline: 1.7312x; 1.3937x over previous
import functools

import jax
import jax.numpy as jnp
from jax import lax
from jax.experimental import pallas as pl
from jax.experimental.pallas import tpu as pltpu
from jax.experimental.pallas import tpu_sc as plsc

B = 4096
L = 200
D = 32
V = 1000000
NW = 32
CHUNK = 512
NQ = B // CHUNK
ITEMS = L * NQ
PER_W = ITEMS // NW
NG = CHUNK // 128
VFULL = V // 128
VMAIN = (VFULL // NW) * NW
KMAIN = VMAIN // NW
VTAIL = V - VFULL * 128

_mesh = plsc.VectorSubcoreMesh(core_axis_name="c", subcore_axis_name="s")


@functools.partial(
    pl.kernel,
    out_type=(
        jax.ShapeDtypeStruct((V // 4, 128), jnp.float32),
        jax.ShapeDtypeStruct((L, B // 128, 128), jnp.int32),
    ),
    mesh=_mesh,
    scratch_types=[
        pltpu.VMEM((2, 32, 129), jnp.float32),
        pltpu.VMEM((2, 32, 128), jnp.float32),
        pltpu.VMEM((PER_W // 2, 8, 128), jnp.int32),
        pltpu.SemaphoreType.DMA,
        pltpu.SemaphoreType.DMA,
        pltpu.SemaphoreType.DMA,
        pltpu.SemaphoreType.DMA,
        pltpu.SemaphoreType.DMA,
        pltpu.SemaphoreType.DMA,
    ],
    compiler_params=pltpu.CompilerParams(use_tc_tiling_on_sc=True,
                                         needs_layout_passes=False),
)
def _prep(xt_hbm, tabt_hbm, tail_hbm, trm_hbm, xi_hbm,
          tin_v, tout_v, xb_v, isemA, isemB, osemA, osemB, xisem, xosem):
    wid = lax.axis_index("s") * 2 + lax.axis_index("c")
    iota = lax.iota(jnp.int32, 16)

    def transpose_block(src, dst):
        def body(t, _):
            col = jnp.full((16,), t, jnp.int32)
            lo = plsc.load_gather(src, [iota, col])
            hi = plsc.load_gather(src, [iota + 16, col])
            r = t // 4
            c0 = (t % 4) * 32
            dst[r, pl.ds(c0, 16)] = lo
            dst[r, pl.ds(c0 + 16, 16)] = hi
            return 0
        lax.fori_loop(0, 128, body, 0, unroll=8)

    def start_in(k, buf):
        b = wid + NW * k
        pltpu.async_copy(tabt_hbm.at[:, pl.ds(b * 128, 128)],
                         tin_v.at[buf, :, pl.ds(0, 128)],
                         isemA if buf == 0 else isemB)

    def wait_in(buf):
        pltpu.make_async_copy(tabt_hbm.at[:, pl.ds(0, 128)],
                              tin_v.at[buf, :, pl.ds(0, 128)],
                              isemA if buf == 0 else isemB).wait()

    def start_out(k, buf):
        b = wid + NW * k
        pltpu.async_copy(tout_v.at[buf], trm_hbm.at[pl.ds(b * 32, 32)],
                         osemA if buf == 0 else osemB)

    def wait_out(k, buf):
        b = wid + NW * k
        pltpu.make_async_copy(tout_v.at[buf], trm_hbm.at[pl.ds(b * 32, 32)],
                              osemA if buf == 0 else osemB).wait()

    start_in(0, 0)

    def pair_body(kk, carry):
        for par in (0, 1):
            k = kk * 2 + par
            wait_in(par)

            @pl.when(k + 1 < KMAIN)
            def _():
                start_in(k + 1, 1 - par)

            @pl.when(k >= 2)
            def _():
                wait_out(k - 2, par)

            transpose_block(tin_v.at[par], tout_v.at[par])
            start_out(k, par)
        return carry

    lax.fori_loop(0, KMAIN // 2, pair_body, 0)
    wait_out(KMAIN - 2, 0)
    wait_out(KMAIN - 1, 1)

    @pl.when(wid < VFULL - VMAIN)
    def _ragged():
        b = VMAIN + wid
        pltpu.sync_copy(tabt_hbm.at[:, pl.ds(b * 128, 128)],
                        tin_v.at[0, :, pl.ds(0, 128)])
        transpose_block(tin_v.at[0], tout_v.at[0])
        pltpu.sync_copy(tout_v.at[0], trm_hbm.at[pl.ds(b * 32, 32)])

    @pl.when(wid == VFULL - VMAIN)
    def _tail():
        pltpu.sync_copy(tail_hbm, tout_v.at[0, pl.ds(0, VTAIL * D // 128)])
        pltpu.sync_copy(tout_v.at[0, pl.ds(0, VTAIL * D // 128)],
                        trm_hbm.at[pl.ds(VFULL * 32, VTAIL * D // 128)])

    NB = PER_W // 2
    for i in range(NB):
        m = wid * NB + i
        lt = m // (B // 128)
        c = m % (B // 128)
        pltpu.async_copy(xt_hbm.at[pl.ds(lt * 8, 8), pl.ds(c * 128, 128)],
                         xb_v.at[i], xisem)
    for i in range(NB):
        pltpu.make_async_copy(xt_hbm.at[pl.ds(0, 8), pl.ds(0, 128)],
                              xb_v.at[i], xisem).wait()
    for i in range(NB):
        m = wid * NB + i
        lt = m // (B // 128)
        c = m % (B // 128)
        pltpu.async_copy(xb_v.at[i], xi_hbm.at[pl.ds(lt * 8, 8), c], xosem)
    for i in range(NB):
        m = wid * NB + i
        lt = m // (B // 128)
        c = m % (B // 128)
        pltpu.make_async_copy(xb_v.at[i], xi_hbm.at[pl.ds(lt * 8, 8), c],
                              xosem).wait()


@functools.partial(
    pl.kernel,
    out_type=jax.ShapeDtypeStruct((L, D // 8, B // 128, 8, 128), jnp.float32),
    mesh=_mesh,
    scratch_types=[
        pltpu.VMEM((2, NG, 128), jnp.int32),
        pltpu.VMEM((2, CHUNK, D), jnp.float32),
        pltpu.VMEM((2, 4, 128), jnp.float32),
        pltpu.VMEM((2, D // 8, NG, 8, 128), jnp.float32),
        pltpu.SemaphoreType.DMA,
        pltpu.SemaphoreType.DMA,
        pltpu.SemaphoreType.DMA,
        pltpu.SemaphoreType.DMA,
        pltpu.SemaphoreType.DMA,
        pltpu.SemaphoreType.DMA,
    ],
    compiler_params=pltpu.CompilerParams(use_tc_tiling_on_sc=False,
                                         needs_layout_passes=False),
)
def _lookup(xi_hbm, trm_hbm, posb_hbm, out_hbm,
            idx_v, rows_v, pos_v, ob_v,
            isemA, isemB, gsemA, gsemB, osemA, osemB):
    wid = lax.axis_index("s") * 2 + lax.axis_index("c")
    iota = lax.iota(jnp.int32, 16)
    isem = (isemA, isemB)
    gsem = (gsemA, gsemB)
    osem = (osemA, osemB)

    def lq(j):
        m = wid * PER_W + j
        return m // NQ, m % NQ

    def start_inputs(j, par):
        l, q = lq(j)
        pltpu.async_copy(xi_hbm.at[l, pl.ds(q * NG, NG)], idx_v.at[par],
                         isem[par])
        pltpu.async_copy(posb_hbm.at[pl.ds(l * 4, 4)], pos_v.at[par],
                         isem[par])

    def wait_inputs(par):
        pltpu.make_async_copy(xi_hbm.at[0, pl.ds(0, NG)], idx_v.at[par],
                              isem[par]).wait()
        pltpu.make_async_copy(posb_hbm.at[pl.ds(0, 4)], pos_v.at[par],
                              isem[par]).wait()

    def start_gathers(par):
        for k in range(NG):
            pltpu.async_copy(trm_hbm.at[idx_v.at[par, k]],
                             rows_v.at[par, pl.ds(k * 128, 128)], gsem[par])

    def wait_gathers(par):
        pltpu.make_async_copy(trm_hbm.at[pl.ds(0, CHUNK)], rows_v.at[par],
                              gsem[par]).wait()

    def start_write(j, par):
        l, q = lq(j)
        pltpu.async_copy(ob_v.at[par],
                         out_hbm.at[l, :, pl.ds(q * NG, NG)], osem[par])

    def wait_write(j, par):
        l, q = lq(j)
        pltpu.make_async_copy(ob_v.at[par],
                              out_hbm.at[l, :, pl.ds(q * NG, NG)],
                              osem[par]).wait()

    perm_lo = [(iota - (1 << k)) & 15 for k in range(4)]
    perm_hi = [(iota + (1 << k)) & 15 for k in range(4)]
    masks = [(iota & (1 << k)) == 0 for k in range(4)]
    _dnums = lax.GatherDimensionNumbers(
        offset_dims=(), collapsed_slice_dims=(0,), start_index_map=(0,))

    def _shift(v, perm):
        return lax.gather(v, perm[:, None], _dnums, (1,),
                          mode=lax.GatherScatterMode.PROMISE_IN_BOUNDS)

    def compute(par):
        for dh in range(2):
            def g_body(g, _, dh=dh):
                t0 = g * 16
                cur = [rows_v[par, t0 + i, pl.ds(dh * 16, 16)]
                       for i in range(16)]
                for k in range(4):
                    m = 1 << k
                    nxt = [None] * 16
                    for i in range(16):
                        if i & m == 0:
                            sh = _shift(cur[i + m], perm_lo[k])
                            nxt[i] = jnp.where(masks[k], cur[i], sh)
                        else:
                            sh = _shift(cur[i - m], perm_hi[k])
                            nxt[i] = jnp.where(masks[k], sh, cur[i])
                    cur = nxt
                cp = g // 8
                mm = g % 8
                for j in range(16):
                    d = dh * 16 + j
                    splat = pos_v[par, d // 8, pl.ds((d % 8) * 16, 16)]
                    ob_v[par, d // 8, cp, d % 8, pl.ds(mm * 16, 16)] = (
                        cur[j] + splat)
                return 0

            lax.fori_loop(0, CHUNK // 16, g_body, 0)

    start_inputs(0, 0)
    wait_inputs(0)
    start_gathers(0)
    start_inputs(1, 1)

    def pair_body(kk, carry):
        for par in (0, 1):
            j = kk * 2 + par

            @pl.when(j + 1 < PER_W)
            def _():
                wait_inputs(1 - par)
                start_gathers(1 - par)

            wait_gathers(par)

            @pl.when(j >= 2)
            def _():
                wait_write(j - 2, par)

            compute(par)
            start_write(j, par)

            @pl.when(j + 2 < PER_W)
            def _():
                start_inputs(j + 2, par)
        return carry

    lax.fori_loop(0, PER_W // 2, pair_body, 0)
    wait_write(PER_W - 2, 0)
    wait_write(PER_W - 1, 1)


def kernel(x, embedding_table, possitional_emb):
    xt = x.T.astype(jnp.int32)
    tabt = embedding_table.T
    tail = embedding_table[VFULL * 128:].reshape(VTAIL * D // 128, 128)
    posb = (jnp.broadcast_to(possitional_emb[:, :, None], (L, D, 16))
            .reshape(L * 4, 128))
    trm, xi = _prep(xt, tabt, tail)
    out5 = _lookup(xi, trm.reshape(V, D), posb)
    return out5.transpose(2, 4, 0, 1, 3).reshape(B, L, D)
